# baseline (speedup 1.0000x reference)
import functools

import jax
import jax.numpy as jnp
from jax import lax
from jax.experimental import pallas as pl
from jax.experimental.pallas import tpu as pltpu
from jax.experimental.pallas import tpu_sc as plsc

NUM_EXPERTS = 16
TOP_K = 2
HIDDEN = 2048
INTER = 1024
T = 2048
A = T * TOP_K
BM = 256
G = A // BM + NUM_EXPERTS
NPAD = G * BM
NW = 32
L = 16


def _router_body(x_ref, gw_ref, logits_ref, topw_ref, sel_ref):
    x = x_ref[...]
    logits = lax.dot_general(x, gw_ref[...], (((1,), (1,)), ((), ())),
                             preferred_element_type=jnp.float32)
    logits_ref[...] = logits
    p = jax.nn.softmax(logits, axis=-1)
    w1 = jnp.max(p, axis=-1)
    a1 = jnp.argmax(p, axis=-1).astype(jnp.int32)
    lanes = lax.broadcasted_iota(jnp.int32, p.shape, 1)
    p2 = jnp.where(lanes == a1[:, None], -jnp.inf, p)
    w2 = jnp.max(p2, axis=-1)
    a2 = jnp.argmax(p2, axis=-1).astype(jnp.int32)
    topw_ref[...] = jnp.stack([w1, w2], axis=1)
    sel_ref[...] = jnp.stack([a1, a2], axis=1)


def _bcast(v, i):
    idx = jnp.full((L,), i, jnp.int32)
    return v.at[idx].get(mode="promise_in_bounds")


def _dispatch_body(sel_hbm, w_hbm, x_hbm, xs_hbm, sw_hbm, be_hbm, bv_hbm,
                   posk_hbm, nact_hbm, selv, wv, stok, swb, cpos, posall,
                   histsv, hloc, bebuf, bvbuf, rowbuf2, semg0, semg1,
                   hists_sh, pos_sh):
    cid = lax.axis_index("c")
    sid = lax.axis_index("s")
    wid = cid * 16 + sid
    lane = lax.iota(jnp.int32, L)
    apw = A // NW
    j0g = sid * 2 * apw

    pltpu.sync_copy(sel_hbm.at[pl.ds(j0g, 2 * apw)], selv)
    pltpu.sync_copy(w_hbm, wv)

    for r in range(2):
        h = jnp.zeros((L,), jnp.int32)
        for v in range(apw // L):
            ev = selv[pl.ds((r * apw + v * L), L)]
            for e in range(NUM_EXPERTS):
                c = plsc.cumsum((ev == e).astype(jnp.int32))
                h = h + jnp.where(lane == e, _bcast(c, L - 1), 0)
        hloc[r, :] = h
    pltpu.sync_copy(hloc, hists_sh.at[pl.ds(2 * sid, 2)])
    plsc.subcore_barrier()
    pltpu.sync_copy(hists_sh, histsv)

    counts = jnp.zeros((L,), jnp.int32)
    prefix = jnp.zeros((L,), jnp.int32)
    for r in range(NW):
        row = histsv[r, :]
        counts = counts + row
        prefix = prefix + jnp.where(r < 2 * sid, row, 0)

    padded = jnp.bitwise_and(counts + (BM - 1), -BM)
    incl = plsc.cumsum(padded)
    pstart = incl - padded
    nblk = lax.shift_right_arithmetic(padded, 8)
    sblk = lax.shift_right_arithmetic(pstart, 8)

    last_e = jnp.max(jnp.where(counts > 0, lane, 0))
    for gv in range(G // L):
        gvec = lane + L * gv
        bev = jnp.zeros((L,), jnp.int32)
        bvv = jnp.zeros((L,), jnp.int32)
        for e in range(NUM_EXPERTS):
            sb = _bcast(sblk, e)
            nb = _bcast(nblk, e)
            m = (gvec >= sb) & (gvec < sb + nb)
            bev = jnp.where(m, e, bev)
            bvv = jnp.where(m, 1, bvv)
        bev = jnp.where(bvv == 1, bev, last_e)
        bebuf[pl.ds(L * gv, L)] = bev
        bvbuf[pl.ds(L * gv, L)] = bvv

    @pl.when(wid == 0)
    def _():
        pltpu.sync_copy(bebuf, be_hbm)
        pltpu.sync_copy(bvbuf, bv_hbm)

    def pos_body(i, base):
        ev = selv[pl.ds(i * L, L)]
        rank = jnp.zeros((L,), jnp.int32)
        upd = jnp.zeros((L,), jnp.int32)
        for e in range(NUM_EXPERTS):
            m = ev == e
            ci = plsc.cumsum(m.astype(jnp.int32))
            rank = rank + jnp.where(m, ci - 1, 0)
            upd = upd + jnp.where(lane == e, _bcast(ci, L - 1), 0)
        posv = base.at[ev].get(mode="promise_in_bounds") + rank
        cpos[pl.ds(i * L, L)] = posv
        return base + upd

    lax.fori_loop(0, 2 * apw // L, pos_body, pstart + prefix)
    pltpu.sync_copy(cpos, pos_sh.at[pl.ds(j0g, 2 * apw)])

    @pl.when(cid == 0)
    def _():
        tpw = apw
        for v in range(2 * apw // L):
            posv = cpos[pl.ds(v * L, L)]
            lj = v * L + lane
            pidx = lax.shift_right_arithmetic(lj, 1) + jnp.bitwise_and(lj, 1) * tpw
            plsc.store_scatter(posall, [pidx], posv,
                               mask=jnp.ones((L,), jnp.bool_))
        pltpu.sync_copy(posall.at[pl.ds(0, tpw)],
                        posk_hbm.at[0, pl.ds(sid * tpw, tpw)])
        pltpu.sync_copy(posall.at[pl.ds(tpw, tpw)],
                        posk_hbm.at[1, pl.ds(sid * tpw, tpw)])

    plsc.subcore_barrier()
    pltpu.sync_copy(pos_sh, posall)

    mybase = wid * BM
    for i in range(BM // L):
        stok[pl.ds(i * L, L)] = jnp.zeros((L,), jnp.int32)
        swb[pl.ds(i * L, L)] = jnp.zeros((L,), jnp.float32)

    def slice_body(i, carry):
        pv = posall[pl.ds(i * L, L)]
        jj = i * L + lane
        tokv = lax.shift_right_arithmetic(jj, 1)
        wvv = wv[pl.ds(i * L, L)]
        lpos = pv - mybase
        inb = (lpos >= 0) & (lpos < BM)
        lposc = jnp.clip(lpos, 0, BM - 1)
        plsc.store_scatter(stok, [lposc], tokv, mask=inb)
        plsc.store_scatter(swb, [lposc], wvv, mask=inb)
        return carry

    lax.fori_loop(0, A // L, slice_body, 0)
    pltpu.sync_copy(swb, sw_hbm.at[pl.ds(mybase, BM)])

    lo = wid * BM
    seg_end = pstart + counts
    nvalid_vec = jnp.clip(jnp.minimum(seg_end, lo + BM) - jnp.maximum(pstart, lo),
                          0, BM)
    nvalid = jnp.sum(nvalid_vec)

    @pl.when(wid == 0)
    def _():
        tb = lax.shift_right_arithmetic(_bcast(incl, L - 1), 8)
        hloc[0, :] = tb
        pltpu.sync_copy(hloc.at[0, pl.ds(0, 8)], nact_hbm)

    waves = BM // L
    sems = [semg0, semg1]
    copies = []
    for w in range(waves):
        ivec = stok[pl.ds(w * L, L)]
        copies.append(pltpu.make_async_copy(
            x_hbm.at[ivec], rowbuf2.at[w % 2], sems[w % 2]))
    for w in range(waves):
        @pl.when(w * L < nvalid)
        def _(w=w):
            copies[w].start()

        if w > 0:
            @pl.when((w - 1) * L < nvalid)
            def _(w=w):
                copies[w - 1].wait()
                pltpu.sync_copy(rowbuf2.at[(w - 1) % 2],
                                xs_hbm.at[pl.ds(mybase + (w - 1) * L, L)])

    @pl.when((waves - 1) * L < nvalid)
    def _():
        copies[waves - 1].wait()
        pltpu.sync_copy(rowbuf2.at[(waves - 1) % 2],
                        xs_hbm.at[pl.ds(mybase + BM - L, L)])


_dispatch = functools.partial(
    pl.kernel,
    out_type=(
        jax.ShapeDtypeStruct((NPAD, HIDDEN), jnp.float32),
        jax.ShapeDtypeStruct((NPAD,), jnp.float32),
        jax.ShapeDtypeStruct((G,), jnp.int32),
        jax.ShapeDtypeStruct((G,), jnp.int32),
        jax.ShapeDtypeStruct((TOP_K, T), jnp.int32),
        jax.ShapeDtypeStruct((8,), jnp.int32),
    ),
    mesh=plsc.VectorSubcoreMesh(core_axis_name="c", subcore_axis_name="s"),
    compiler_params=pltpu.CompilerParams(needs_layout_passes=False),
    scratch_types=[
        pltpu.VMEM((2 * A // NW,), jnp.int32),
        pltpu.VMEM((A,), jnp.float32),
        pltpu.VMEM((BM,), jnp.int32),
        pltpu.VMEM((BM,), jnp.float32),
        pltpu.VMEM((2 * A // NW,), jnp.int32),
        pltpu.VMEM((A,), jnp.int32),
        pltpu.VMEM((NW, L), jnp.int32),
        pltpu.VMEM((2, L), jnp.int32),
        pltpu.VMEM((G,), jnp.int32),
        pltpu.VMEM((G,), jnp.int32),
        pltpu.VMEM((2, L, HIDDEN), jnp.float32),
        pltpu.SemaphoreType.DMA,
        pltpu.SemaphoreType.DMA,
        pltpu.VMEM_SHARED((NW, L), jnp.int32),
        pltpu.VMEM_SHARED((A,), jnp.int32),
    ],
)(_dispatch_body)


def _gmm_body(be_ref, bv_ref, na_ref, xs_ref, sw_ref, gw_ref, uw_ref, dw_ref,
              ys_ref):
    g = pl.program_id(0)
    k = pl.program_id(1)

    @pl.when(bv_ref[g] == 1)
    def _():
        xv = xs_ref[...]
        gg = lax.dot_general(xv, gw_ref[0], (((1,), (1,)), ((), ())),
                             preferred_element_type=jnp.float32)
        uu = lax.dot_general(xv, uw_ref[0], (((1,), (1,)), ((), ())),
                             preferred_element_type=jnp.float32)
        h = (gg * lax.logistic(gg)) * uu
        o = lax.dot_general(h, dw_ref[0], (((1,), (1,)), ((), ())),
                            preferred_element_type=jnp.float32)

        @pl.when(k == 0)
        def _():
            ys_ref[...] = o * sw_ref[...]

        @pl.when(k == 1)
        def _():
            ys_ref[...] += o * sw_ref[...]


def _scatter_body(ys_hbm, posk_hbm, out_hbm, posA, posB, bufA, bufB, semA, semB):
    wid = lax.axis_index("c") * 16 + lax.axis_index("s")
    tpw = T // NW
    t0 = wid * tpw
    pltpu.sync_copy(posk_hbm.at[0, pl.ds(t0, tpw)], posA)
    pltpu.sync_copy(posk_hbm.at[1, pl.ds(t0, tpw)], posB)
    for w in range(tpw // L):
        ia = posA[pl.ds(w * L, L)]
        ib = posB[pl.ds(w * L, L)]
        ca = pltpu.async_copy(ys_hbm.at[ia], bufA, semA)
        cb = pltpu.async_copy(ys_hbm.at[ib], bufB, semB)
        ca.wait()
        cb.wait()

        def add_body(i, _):
            for u in range(4):
                ii = i * 4 + u
                r = lax.shift_right_arithmetic(ii, 7)
                c = jnp.bitwise_and(ii, 127) * L
                plsc.addupdate(bufA.at[r, pl.ds(c, L)], bufB[r, pl.ds(c, L)])
            return 0

        lax.fori_loop(0, L * HIDDEN // (L * 4), add_body, 0)
        pltpu.sync_copy(bufA, out_hbm.at[pl.ds(t0 + w * L, L)])


_scatter = functools.partial(
    pl.kernel,
    out_type=jax.ShapeDtypeStruct((T, HIDDEN), jnp.float32),
    mesh=plsc.VectorSubcoreMesh(core_axis_name="c", subcore_axis_name="s"),
    compiler_params=pltpu.CompilerParams(needs_layout_passes=False),
    scratch_types=[
        pltpu.VMEM((T // NW,), jnp.int32),
        pltpu.VMEM((T // NW,), jnp.int32),
        pltpu.VMEM((L, HIDDEN), jnp.float32),
        pltpu.VMEM((L, HIDDEN), jnp.float32),
        pltpu.SemaphoreType.DMA,
        pltpu.SemaphoreType.DMA,
    ],
)(_scatter_body)


@jax.jit
def _moe(x, gate_w, gate_proj_w, up_proj_w, down_proj_w):
    logits, topw, sel = pl.pallas_call(
        _router_body,
        out_shape=(
            jax.ShapeDtypeStruct((T, NUM_EXPERTS), jnp.float32),
            jax.ShapeDtypeStruct((T, TOP_K), jnp.float32),
            jax.ShapeDtypeStruct((T, TOP_K), jnp.int32),
        ),
    )(x, gate_w)

    xs, sw, be, bv, posk, nact = _dispatch(sel.reshape(-1), topw.reshape(-1), x)

    ys = pl.pallas_call(
        _gmm_body,
        grid_spec=pltpu.PrefetchScalarGridSpec(
            num_scalar_prefetch=3,
            grid=(G, 2),
            in_specs=[
                pl.BlockSpec((BM, HIDDEN),
                             lambda g, k, be, bv, na: (jnp.minimum(g, na[0] - 1), 0)),
                pl.BlockSpec((BM, 1),
                             lambda g, k, be, bv, na: (jnp.minimum(g, na[0] - 1), 0)),
                pl.BlockSpec((1, INTER // 2, HIDDEN),
                             lambda g, k, be, bv, na: (be[g], k, 0)),
                pl.BlockSpec((1, INTER // 2, HIDDEN),
                             lambda g, k, be, bv, na: (be[g], k, 0)),
                pl.BlockSpec((1, HIDDEN, INTER // 2),
                             lambda g, k, be, bv, na: (be[g], 0, k)),
            ],
            out_specs=pl.BlockSpec(
                (BM, HIDDEN),
                lambda g, k, be, bv, na: (jnp.minimum(g, na[0] - 1), 0)),
        ),
        out_shape=jax.ShapeDtypeStruct((NPAD, HIDDEN), jnp.float32),
        compiler_params=pltpu.CompilerParams(
            dimension_semantics=("arbitrary", "arbitrary"),
        ),
    )(be, bv, nact, xs, sw.reshape(NPAD, 1), gate_proj_w, up_proj_w,
      down_proj_w)

    out = _scatter(ys, posk)
    return out, logits


def kernel(hidden_states, gate_w, gate_proj_w, up_proj_w, down_proj_w):
    b, s, d = hidden_states.shape
    x = hidden_states.reshape(-1, d)
    out, logits = _moe(x, gate_w, gate_proj_w, up_proj_w, down_proj_w)
    return out.reshape(b, s, d), logits


# speedup vs baseline: 1.3340x; 1.3340x over previous
import functools

import jax
import jax.numpy as jnp
from jax import lax
from jax.experimental import pallas as pl
from jax.experimental.pallas import tpu as pltpu
from jax.experimental.pallas import tpu_sc as plsc

NUM_EXPERTS = 16
TOP_K = 2
HIDDEN = 2048
INTER = 1024
T = 2048
A = T * TOP_K
BM = 256
G = A // BM + NUM_EXPERTS
NPAD = G * BM
NW = 32
L = 16


def _router_body(x_ref, gw_ref, logits_ref, topw_ref, sel_ref):
    x = x_ref[...]
    logits = lax.dot_general(x, gw_ref[...], (((1,), (1,)), ((), ())),
                             preferred_element_type=jnp.float32)
    logits_ref[...] = logits
    p = jax.nn.softmax(logits, axis=-1)
    w1 = jnp.max(p, axis=-1)
    a1 = jnp.argmax(p, axis=-1).astype(jnp.int32)
    lanes = lax.broadcasted_iota(jnp.int32, p.shape, 1)
    p2 = jnp.where(lanes == a1[:, None], -jnp.inf, p)
    w2 = jnp.max(p2, axis=-1)
    a2 = jnp.argmax(p2, axis=-1).astype(jnp.int32)
    topw_ref[...] = jnp.stack([w1, w2], axis=1)
    sel_ref[...] = jnp.stack([a1, a2], axis=1)


def _bcast(v, i):
    idx = jnp.full((L,), i, jnp.int32)
    return v.at[idx].get(mode="promise_in_bounds")


def _dispatch_body(sel_hbm, w_hbm, x_hbm, xs_hbm, sw_hbm, be_hbm, bv_hbm,
                   posk_hbm, nact_hbm, selv, wv, stok, swb, cpos, posall,
                   histsv, hloc, bebuf, bvbuf, rowbuf2, semg0, semg1,
                   hists_sh, pos_sh):
    cid = lax.axis_index("c")
    sid = lax.axis_index("s")
    wid = cid * 16 + sid
    lane = lax.iota(jnp.int32, L)
    apw = A // NW
    j0g = sid * 2 * apw

    pltpu.sync_copy(sel_hbm.at[pl.ds(j0g, 2 * apw)], selv)
    pltpu.sync_copy(w_hbm, wv)

    for r in range(2):
        h = jnp.zeros((L,), jnp.int32)
        for v in range(apw // L):
            ev = selv[pl.ds((r * apw + v * L), L)]
            for e in range(NUM_EXPERTS):
                c = plsc.cumsum((ev == e).astype(jnp.int32))
                h = h + jnp.where(lane == e, _bcast(c, L - 1), 0)
        hloc[r, :] = h
    pltpu.sync_copy(hloc, hists_sh.at[pl.ds(2 * sid, 2)])
    plsc.subcore_barrier()
    pltpu.sync_copy(hists_sh, histsv)

    counts = jnp.zeros((L,), jnp.int32)
    prefix = jnp.zeros((L,), jnp.int32)
    for r in range(NW):
        row = histsv[r, :]
        counts = counts + row
        prefix = prefix + jnp.where(r < 2 * sid, row, 0)

    padded = jnp.bitwise_and(counts + (BM - 1), -BM)
    incl = plsc.cumsum(padded)
    pstart = incl - padded
    nblk = lax.shift_right_arithmetic(padded, 8)
    sblk = lax.shift_right_arithmetic(pstart, 8)

    last_e = jnp.max(jnp.where(counts > 0, lane, 0))
    for gv in range(G // L):
        gvec = lane + L * gv
        bev = jnp.zeros((L,), jnp.int32)
        bvv = jnp.zeros((L,), jnp.int32)
        for e in range(NUM_EXPERTS):
            sb = _bcast(sblk, e)
            nb = _bcast(nblk, e)
            m = (gvec >= sb) & (gvec < sb + nb)
            bev = jnp.where(m, e, bev)
            bvv = jnp.where(m, 1, bvv)
        bev = jnp.where(bvv == 1, bev, last_e)
        bebuf[pl.ds(L * gv, L)] = bev
        bvbuf[pl.ds(L * gv, L)] = bvv

    @pl.when(wid == 0)
    def _():
        pltpu.sync_copy(bebuf, be_hbm)
        pltpu.sync_copy(bvbuf, bv_hbm)

    def pos_body(i, base):
        ev = selv[pl.ds(i * L, L)]
        rank = jnp.zeros((L,), jnp.int32)
        upd = jnp.zeros((L,), jnp.int32)
        for e in range(NUM_EXPERTS):
            m = ev == e
            ci = plsc.cumsum(m.astype(jnp.int32))
            rank = rank + jnp.where(m, ci - 1, 0)
            upd = upd + jnp.where(lane == e, _bcast(ci, L - 1), 0)
        posv = base.at[ev].get(mode="promise_in_bounds") + rank
        cpos[pl.ds(i * L, L)] = posv
        return base + upd

    lax.fori_loop(0, 2 * apw // L, pos_body, pstart + prefix)
    pltpu.sync_copy(cpos, pos_sh.at[pl.ds(j0g, 2 * apw)])

    @pl.when(cid == 0)
    def _():
        tpw = apw
        for v in range(2 * apw // L):
            posv = cpos[pl.ds(v * L, L)]
            lj = v * L + lane
            pidx = lax.shift_right_arithmetic(lj, 1) + jnp.bitwise_and(lj, 1) * tpw
            plsc.store_scatter(posall, [pidx], posv,
                               mask=jnp.ones((L,), jnp.bool_))
        pltpu.sync_copy(posall.at[pl.ds(0, tpw)],
                        posk_hbm.at[0, pl.ds(sid * tpw, tpw)])
        pltpu.sync_copy(posall.at[pl.ds(tpw, tpw)],
                        posk_hbm.at[1, pl.ds(sid * tpw, tpw)])

    plsc.subcore_barrier()
    pltpu.sync_copy(pos_sh, posall)

    mybase = wid * BM
    for i in range(BM // L):
        stok[pl.ds(i * L, L)] = jnp.zeros((L,), jnp.int32)
        swb[pl.ds(i * L, L)] = jnp.zeros((L,), jnp.float32)

    def slice_body(i, carry):
        pv = posall[pl.ds(i * L, L)]
        jj = i * L + lane
        tokv = lax.shift_right_arithmetic(jj, 1)
        wvv = wv[pl.ds(i * L, L)]
        lpos = pv - mybase
        inb = (lpos >= 0) & (lpos < BM)
        lposc = jnp.clip(lpos, 0, BM - 1)
        plsc.store_scatter(stok, [lposc], tokv, mask=inb)
        plsc.store_scatter(swb, [lposc], wvv, mask=inb)
        return carry

    lax.fori_loop(0, A // L, slice_body, 0)
    pltpu.sync_copy(swb, sw_hbm.at[pl.ds(mybase, BM)])

    lo = wid * BM
    seg_end = pstart + counts
    nvalid_vec = jnp.clip(jnp.minimum(seg_end, lo + BM) - jnp.maximum(pstart, lo),
                          0, BM)
    nvalid = jnp.sum(nvalid_vec)

    @pl.when(wid == 0)
    def _():
        tb = lax.shift_right_arithmetic(_bcast(incl, L - 1), 8)
        hloc[0, :] = tb
        pltpu.sync_copy(hloc.at[0, pl.ds(0, 8)], nact_hbm)

    waves = BM // L
    sems = [semg0, semg1]
    copies = []
    for w in range(waves):
        ivec = stok[pl.ds(w * L, L)]
        copies.append(pltpu.make_async_copy(
            x_hbm.at[ivec], rowbuf2.at[w % 2], sems[w % 2]))
    for w in range(waves):
        @pl.when(w * L < nvalid)
        def _(w=w):
            copies[w].start()

        if w > 0:
            @pl.when((w - 1) * L < nvalid)
            def _(w=w):
                copies[w - 1].wait()
                pltpu.sync_copy(rowbuf2.at[(w - 1) % 2],
                                xs_hbm.at[pl.ds(mybase + (w - 1) * L, L)])

    @pl.when((waves - 1) * L < nvalid)
    def _():
        copies[waves - 1].wait()
        pltpu.sync_copy(rowbuf2.at[(waves - 1) % 2],
                        xs_hbm.at[pl.ds(mybase + BM - L, L)])


_dispatch = functools.partial(
    pl.kernel,
    out_type=(
        jax.ShapeDtypeStruct((NPAD, HIDDEN), jnp.float32),
        jax.ShapeDtypeStruct((NPAD,), jnp.float32),
        jax.ShapeDtypeStruct((G,), jnp.int32),
        jax.ShapeDtypeStruct((G,), jnp.int32),
        jax.ShapeDtypeStruct((TOP_K, T), jnp.int32),
        jax.ShapeDtypeStruct((8,), jnp.int32),
    ),
    mesh=plsc.VectorSubcoreMesh(core_axis_name="c", subcore_axis_name="s"),
    compiler_params=pltpu.CompilerParams(needs_layout_passes=False),
    scratch_types=[
        pltpu.VMEM((2 * A // NW,), jnp.int32),
        pltpu.VMEM((A,), jnp.float32),
        pltpu.VMEM((BM,), jnp.int32),
        pltpu.VMEM((BM,), jnp.float32),
        pltpu.VMEM((2 * A // NW,), jnp.int32),
        pltpu.VMEM((A,), jnp.int32),
        pltpu.VMEM((NW, L), jnp.int32),
        pltpu.VMEM((2, L), jnp.int32),
        pltpu.VMEM((G,), jnp.int32),
        pltpu.VMEM((G,), jnp.int32),
        pltpu.VMEM((2, L, HIDDEN), jnp.float32),
        pltpu.SemaphoreType.DMA,
        pltpu.SemaphoreType.DMA,
        pltpu.VMEM_SHARED((NW, L), jnp.int32),
        pltpu.VMEM_SHARED((A,), jnp.int32),
    ],
)(_dispatch_body)


def _gmm_body(be_ref, bv_ref, na_ref, xs_ref, sw_ref, gw_ref, uw_ref, dw_ref,
              ys_ref):
    g = pl.program_id(0)

    @pl.when(bv_ref[g] == 1)
    def _():
        xv = xs_ref[...]
        gg = lax.dot_general(xv, gw_ref[0], (((1,), (1,)), ((), ())),
                             preferred_element_type=jnp.float32)
        uu = lax.dot_general(xv, uw_ref[0], (((1,), (1,)), ((), ())),
                             preferred_element_type=jnp.float32)
        h = (gg * lax.logistic(gg)) * uu
        o = lax.dot_general(h, dw_ref[0], (((1,), (1,)), ((), ())),
                            preferred_element_type=jnp.float32)
        ys_ref[...] = o * sw_ref[...]


def _scatter_body(ys_hbm, posk_hbm, out_hbm, posA, posB, bufA, bufB, semA, semB):
    wid = lax.axis_index("c") * 16 + lax.axis_index("s")
    tpw = T // NW
    t0 = wid * tpw
    pltpu.sync_copy(posk_hbm.at[0, pl.ds(t0, tpw)], posA)
    pltpu.sync_copy(posk_hbm.at[1, pl.ds(t0, tpw)], posB)
    for w in range(tpw // L):
        ia = posA[pl.ds(w * L, L)]
        ib = posB[pl.ds(w * L, L)]
        ca = pltpu.async_copy(ys_hbm.at[ia], bufA, semA)
        cb = pltpu.async_copy(ys_hbm.at[ib], bufB, semB)
        ca.wait()
        cb.wait()

        def add_body(i, _):
            for u in range(4):
                ii = i * 4 + u
                r = lax.shift_right_arithmetic(ii, 7)
                c = jnp.bitwise_and(ii, 127) * L
                plsc.addupdate(bufA.at[r, pl.ds(c, L)], bufB[r, pl.ds(c, L)])
            return 0

        lax.fori_loop(0, L * HIDDEN // (L * 4), add_body, 0)
        pltpu.sync_copy(bufA, out_hbm.at[pl.ds(t0 + w * L, L)])


_scatter = functools.partial(
    pl.kernel,
    out_type=jax.ShapeDtypeStruct((T, HIDDEN), jnp.float32),
    mesh=plsc.VectorSubcoreMesh(core_axis_name="c", subcore_axis_name="s"),
    compiler_params=pltpu.CompilerParams(needs_layout_passes=False),
    scratch_types=[
        pltpu.VMEM((T // NW,), jnp.int32),
        pltpu.VMEM((T // NW,), jnp.int32),
        pltpu.VMEM((L, HIDDEN), jnp.float32),
        pltpu.VMEM((L, HIDDEN), jnp.float32),
        pltpu.SemaphoreType.DMA,
        pltpu.SemaphoreType.DMA,
    ],
)(_scatter_body)


@jax.jit
def _moe(x, gate_w, gate_proj_w, up_proj_w, down_proj_w):
    logits, topw, sel = pl.pallas_call(
        _router_body,
        out_shape=(
            jax.ShapeDtypeStruct((T, NUM_EXPERTS), jnp.float32),
            jax.ShapeDtypeStruct((T, TOP_K), jnp.float32),
            jax.ShapeDtypeStruct((T, TOP_K), jnp.int32),
        ),
    )(x, gate_w)

    xs, sw, be, bv, posk, nact = _dispatch(sel.reshape(-1), topw.reshape(-1), x)

    ys = pl.pallas_call(
        _gmm_body,
        grid_spec=pltpu.PrefetchScalarGridSpec(
            num_scalar_prefetch=3,
            grid=(G,),
            in_specs=[
                pl.BlockSpec((BM, HIDDEN),
                             lambda g, be, bv, na: (jnp.minimum(g, na[0] - 1), 0)),
                pl.BlockSpec((BM, 1),
                             lambda g, be, bv, na: (jnp.minimum(g, na[0] - 1), 0)),
                pl.BlockSpec((1, INTER, HIDDEN), lambda g, be, bv, na: (be[g], 0, 0)),
                pl.BlockSpec((1, INTER, HIDDEN), lambda g, be, bv, na: (be[g], 0, 0)),
                pl.BlockSpec((1, HIDDEN, INTER), lambda g, be, bv, na: (be[g], 0, 0)),
            ],
            out_specs=pl.BlockSpec(
                (BM, HIDDEN), lambda g, be, bv, na: (jnp.minimum(g, na[0] - 1), 0)),
        ),
        out_shape=jax.ShapeDtypeStruct((NPAD, HIDDEN), jnp.float32),
        compiler_params=pltpu.CompilerParams(
            dimension_semantics=("arbitrary",),
        ),
    )(be, bv, nact, xs, sw.reshape(NPAD, 1), gate_proj_w, up_proj_w,
      down_proj_w)

    out = _scatter(ys, posk)
    return out, logits


def kernel(hidden_states, gate_w, gate_proj_w, up_proj_w, down_proj_w):
    b, s, d = hidden_states.shape
    x = hidden_states.reshape(-1, d)
    out, logits = _moe(x, gate_w, gate_proj_w, up_proj_w, down_proj_w)
    return out.reshape(b, s, d), logits
